# Initial kernel scaffold; baseline (speedup 1.0000x reference)
import jax
import jax.numpy as jnp
from jax.experimental import pallas as pl


def kernel(x, noise, sample_index):
    raise NotImplementedError("write your pallas kernel here")


# speedup vs baseline: 6.2934x; 6.2934x over previous
import jax
import jax.numpy as jnp
from jax.experimental import pallas as pl
from jax.experimental.pallas import tpu as pltpu

SIGMA = 0.2
ROWS, COLS, NIDX = 16384, 1024, 256
BLOCK_R = 512


def _fused_kernel(idx_ref, x_ref, n_ref, o_ref, mask_ref):
    @pl.when(pl.program_id(0) == 0)
    def _build_mask():
        cols = jax.lax.broadcasted_iota(jnp.int32, (NIDX, COLS), 1)
        hit = cols == idx_ref[...]
        mask_ref[...] = jnp.where(jnp.any(hit, axis=0, keepdims=True), 0.0, 1.0)

    o_ref[...] = x_ref[...] * (1.0 + SIGMA * n_ref[...]) * mask_ref[...]


def kernel(x, noise, sample_index):
    idx = sample_index.astype(jnp.int32).reshape(NIDX, 1)
    return pl.pallas_call(
        _fused_kernel,
        grid=(ROWS // BLOCK_R,),
        in_specs=[
            pl.BlockSpec((NIDX, 1), lambda i: (0, 0)),
            pl.BlockSpec((BLOCK_R, COLS), lambda i: (i, 0)),
            pl.BlockSpec((BLOCK_R, COLS), lambda i: (i, 0)),
        ],
        out_specs=pl.BlockSpec((BLOCK_R, COLS), lambda i: (i, 0)),
        out_shape=jax.ShapeDtypeStruct((ROWS, COLS), jnp.float32),
        scratch_shapes=[pltpu.VMEM((1, COLS), jnp.float32)],
    )(idx, x, noise)


# speedup vs baseline: 6.4021x; 1.0173x over previous
import jax
import jax.numpy as jnp
from jax.experimental import pallas as pl
from jax.experimental.pallas import tpu as pltpu

SIGMA = 0.2
ROWS, COLS, NIDX = 16384, 1024, 256
BLOCK_R = 1024


def _fused_kernel(idx_ref, x_ref, n_ref, o_ref, mask_ref):
    @pl.when(pl.program_id(0) == 0)
    def _build_mask():
        cols = jax.lax.broadcasted_iota(jnp.int32, (NIDX, COLS), 1)
        hit = cols == idx_ref[...]
        mask_ref[...] = jnp.where(jnp.any(hit, axis=0, keepdims=True), 0.0, 1.0)

    o_ref[...] = x_ref[...] * (1.0 + SIGMA * n_ref[...]) * mask_ref[...]


def kernel(x, noise, sample_index):
    idx = sample_index.astype(jnp.int32).reshape(NIDX, 1)
    return pl.pallas_call(
        _fused_kernel,
        grid=(ROWS // BLOCK_R,),
        in_specs=[
            pl.BlockSpec((NIDX, 1), lambda i: (0, 0)),
            pl.BlockSpec((BLOCK_R, COLS), lambda i: (i, 0)),
            pl.BlockSpec((BLOCK_R, COLS), lambda i: (i, 0)),
        ],
        out_specs=pl.BlockSpec((BLOCK_R, COLS), lambda i: (i, 0)),
        out_shape=jax.ShapeDtypeStruct((ROWS, COLS), jnp.float32),
        scratch_shapes=[pltpu.VMEM((1, COLS), jnp.float32)],
    )(idx, x, noise)
